# baseline (speedup 1.0000x reference)
import functools

import jax
import jax.numpy as jnp
import numpy as np
from jax import lax
from jax.experimental import pallas as pl
from jax.experimental.pallas import tpu as pltpu
from jax.experimental.pallas import tpu_sc as plsc

_B, _S, _V = 32, 200, 1000
_NBI = _S - 1
_NTRI = _S - 2
_NCH = 13
_VP = 1024

_DUMP_BI = 1_000_000
_TBL_BI = 1_000_016
_DUMP_TRI = 6_368_000
_TBL_TRI = 6_368_016
_DUMP_PR = _B * _B * _NTRI
_TBL_PR = _DUMP_PR + 16

_mesh = plsc.VectorSubcoreMesh(core_axis_name="c", subcore_axis_name="s")


def _wid():
    return lax.axis_index("s") * 2 + lax.axis_index("c")


def _conf_body(lg_ref, out_ref):
    i = pl.program_id(0)
    x = lg_ref[...]
    m = jnp.max(x, axis=1, keepdims=True)
    s = jnp.sum(jnp.exp(x - m), axis=1)
    part = jnp.sum(1.0 / s)

    @pl.when(i == 0)
    def _():
        out_ref[...] = jnp.zeros((1, 1), jnp.float32)

    out_ref[...] += jnp.full((1, 1), part)


def _prep_body(toks_ref, bik_ref, t2k_ref):
    toks = toks_ref[...]
    bi = toks[:, :-1] * _V + toks[:, 1:]
    bik_ref[...] = jnp.concatenate(
        [bi, jnp.full((_B, 9), _DUMP_BI, jnp.int32)], axis=1)
    t2k_ref[...] = jnp.concatenate(
        [toks[:, 2:], jnp.zeros((_B, 10), jnp.int32)], axis=1)


def _stats_body(toks_ref, pbi_ref, ptri_ref, ppr_ref, conf_ref, out_ref,
                counts_ref, pres_ref):
    counts_ref[...] = jnp.zeros((1, _V), jnp.float32)
    iota_v = lax.broadcasted_iota(jnp.int32, (1, _V), 1)

    def hist_body(b, _):
        row = toks_ref[b, :]
        cmp = row[:, None] == iota_v
        counts_ref[...] += jnp.sum(cmp.astype(jnp.float32), axis=0)[None, :]
        pres_ref[pl.ds(b, 1), :] = jnp.any(cmp, axis=0).astype(jnp.float32)[None, :]
        return 0

    lax.fori_loop(0, _B, hist_body, 0)
    counts = counts_ref[0, :]
    total = jnp.sum(counts)
    probs = counts / (total + 1e-08)
    entropy = -jnp.sum(jnp.where(probs > 0, probs * jnp.log(probs + 1e-08), 0.0))
    token_entropy = 1.0 - entropy / np.log(_V)
    distinct1 = jnp.sum((counts > 0).astype(jnp.float32))

    pres = pres_ref[...]
    inter = lax.dot_general(pres, pres, (((1,), (1,)), ((), ())),
                            preferred_element_type=jnp.float32)
    ru = jnp.sum(pres, axis=1)
    r_i = lax.broadcasted_iota(jnp.int32, (_B, _B), 0)
    c_i = lax.broadcasted_iota(jnp.int32, (_B, _B), 1)
    selmask = ((r_i < 10) & (r_i != c_i)).astype(jnp.float32)
    overlaps = inter / jnp.maximum(ru, 1.0)[:, None]
    self_bleu = jnp.sum(overlaps * selmask) / (10 * (_B - 1))

    u_bi = jnp.sum(pbi_ref[...]).astype(jnp.float32)
    u_tri = jnp.sum(ptri_ref[...]).astype(jnp.float32)
    u_pr = jnp.sum(ppr_ref[...]).astype(jnp.float32)

    repetition = 1.0 - u_pr / (_B * _NTRI)
    d1 = distinct1 / (_B * _S)
    d2 = u_bi / (_B * _NBI)
    d3 = u_tri / (_B * _NTRI)
    ngram_diversity = ((1.0 - d1) + (1.0 - d2) + (1.0 - d3)) / 3.0

    avg_conf = jnp.sum(conf_ref[...]) / (_B * _S)
    overconfidence = jnp.maximum(avg_conf - 0.85, 0.0) * 2.0

    total_loss = (0.25 * ngram_diversity + 0.2 * token_entropy + 0.2 * self_bleu
                  + 0.2 * repetition + 0.15 * overconfidence)

    out_ref[...] = jnp.stack([ngram_diversity, token_entropy, self_bleu,
                              repetition, overconfidence, total_loss])[None, :]


@functools.partial(
    pl.kernel, mesh=_mesh,
    out_type=jax.ShapeDtypeStruct((_TBL_BI,), jnp.int32),
    scratch_types=[pltpu.VMEM((_NCH, 16), jnp.int32),
                   pltpu.VMEM((_NCH, 16), jnp.int32),
                   pltpu.SemaphoreType.DMA])
def _sc_bi_scatter(bik_hbm, tbl_hbm, key_v, val_v, sem):
    b = _wid()
    pltpu.sync_copy(bik_hbm.at[b], key_v)
    iota = lax.iota(jnp.int32, 16)
    for c in range(_NCH):
        val_v[c, :] = b * _NBI + c * 16 + iota
    cps = [pltpu.async_copy(val_v.at[c], tbl_hbm.at[key_v.at[c]], sem)
           for c in range(_NCH)]
    for cp in cps:
        cp.wait()


@functools.partial(
    pl.kernel, mesh=_mesh,
    out_type=[jax.ShapeDtypeStruct((_TBL_TRI,), jnp.int32),
              jax.ShapeDtypeStruct((_B, 16), jnp.int32),
              jax.ShapeDtypeStruct((_B, _NCH, 16), jnp.int32)],
    scratch_types=[pltpu.VMEM((_NCH, 16), jnp.int32),
                   pltpu.VMEM((_NCH, 16), jnp.int32),
                   pltpu.VMEM((_NCH, 16), jnp.int32),
                   pltpu.VMEM((_NCH, 16), jnp.int32),
                   pltpu.VMEM((_NCH, 16), jnp.int32),
                   pltpu.VMEM((16,), jnp.int32),
                   pltpu.SemaphoreType.DMA])
def _sc_tri_scatter(bik_hbm, t2k_hbm, tblbi_hbm,
                    tbltri_hbm, part_hbm, keys3_hbm,
                    key_v, t2_v, g_v, val_v, k3_v, acc_v, sem):
    b = _wid()
    pltpu.sync_copy(bik_hbm.at[b], key_v)
    pltpu.sync_copy(t2k_hbm.at[b], t2_v)
    cps = [pltpu.async_copy(tblbi_hbm.at[key_v.at[c]], g_v.at[c], sem)
           for c in range(_NCH)]
    for cp in cps:
        cp.wait()
    iota = lax.iota(jnp.int32, 16)
    one = jnp.ones((16,), jnp.int32)
    zero = jnp.zeros((16,), jnp.int32)
    dump3 = jnp.full((16,), _DUMP_TRI, jnp.int32)
    acc = zero
    for c in range(_NCH):
        s_c = c * 16 + iota
        g_c = g_v[c, :]
        j_c = b * _NBI + s_c
        acc = acc + jnp.where((s_c < _NBI) & (g_c == j_c), one, zero)
        k3_v[c, :] = jnp.where(s_c < _NTRI, g_c * _V + t2_v[c, :], dump3)
        val_v[c, :] = b * _NTRI + s_c
    acc_v[...] = acc
    pltpu.sync_copy(acc_v, part_hbm.at[b])
    pltpu.sync_copy(k3_v, keys3_hbm.at[b])
    cps = [pltpu.async_copy(val_v.at[c], tbltri_hbm.at[k3_v.at[c]], sem)
           for c in range(_NCH)]
    for cp in cps:
        cp.wait()


@functools.partial(
    pl.kernel, mesh=_mesh,
    out_type=[jax.ShapeDtypeStruct((_TBL_PR,), jnp.int32),
              jax.ShapeDtypeStruct((_B, 16), jnp.int32),
              jax.ShapeDtypeStruct((_B, _NCH, 16), jnp.int32)],
    scratch_types=[pltpu.VMEM((_NCH, 16), jnp.int32),
                   pltpu.VMEM((_NCH, 16), jnp.int32),
                   pltpu.VMEM((_NCH, 16), jnp.int32),
                   pltpu.VMEM((_NCH, 16), jnp.int32),
                   pltpu.VMEM((16,), jnp.int32),
                   pltpu.SemaphoreType.DMA])
def _sc_pr_scatter(keys3_hbm, tbltri_hbm,
                   tblpr_hbm, part_hbm, keys4_hbm,
                   k3_v, h_v, val_v, k4_v, acc_v, sem):
    b = _wid()
    pltpu.sync_copy(keys3_hbm.at[b], k3_v)
    cps = [pltpu.async_copy(tbltri_hbm.at[k3_v.at[c]], h_v.at[c], sem)
           for c in range(_NCH)]
    for cp in cps:
        cp.wait()
    iota = lax.iota(jnp.int32, 16)
    one = jnp.ones((16,), jnp.int32)
    zero = jnp.zeros((16,), jnp.int32)
    dump4 = jnp.full((16,), _DUMP_PR, jnp.int32)
    acc = zero
    for c in range(_NCH):
        s_c = c * 16 + iota
        h_c = h_v[c, :]
        j3_c = b * _NTRI + s_c
        valid = s_c < _NTRI
        acc = acc + jnp.where(valid & (h_c == j3_c), one, zero)
        k4_v[c, :] = jnp.where(valid, b * (_B * _NTRI) + h_c, dump4)
        val_v[c, :] = j3_c
    acc_v[...] = acc
    pltpu.sync_copy(acc_v, part_hbm.at[b])
    pltpu.sync_copy(k4_v, keys4_hbm.at[b])
    cps = [pltpu.async_copy(val_v.at[c], tblpr_hbm.at[k4_v.at[c]], sem)
           for c in range(_NCH)]
    for cp in cps:
        cp.wait()


@functools.partial(
    pl.kernel, mesh=_mesh,
    out_type=jax.ShapeDtypeStruct((_B, 16), jnp.int32),
    scratch_types=[pltpu.VMEM((_NCH, 16), jnp.int32),
                   pltpu.VMEM((_NCH, 16), jnp.int32),
                   pltpu.VMEM((16,), jnp.int32),
                   pltpu.SemaphoreType.DMA])
def _sc_pr_gather(keys4_hbm, tblpr_hbm, part_hbm, k4_v, p_v, acc_v, sem):
    b = _wid()
    pltpu.sync_copy(keys4_hbm.at[b], k4_v)
    cps = [pltpu.async_copy(tblpr_hbm.at[k4_v.at[c]], p_v.at[c], sem)
           for c in range(_NCH)]
    for cp in cps:
        cp.wait()
    iota = lax.iota(jnp.int32, 16)
    one = jnp.ones((16,), jnp.int32)
    zero = jnp.zeros((16,), jnp.int32)
    acc = zero
    for c in range(_NCH):
        s_c = c * 16 + iota
        j3_c = b * _NTRI + s_c
        acc = acc + jnp.where((s_c < _NTRI) & (p_v[c, :] == j3_c), one, zero)
    acc_v[...] = acc
    pltpu.sync_copy(acc_v, part_hbm.at[b])


@jax.jit
def _run(toks, logits):
    toks = toks.astype(jnp.int32)
    lg2 = logits.reshape(_B * _S, _V)
    rows = 800
    conf = pl.pallas_call(
        _conf_body,
        grid=(_B * _S // rows,),
        in_specs=[pl.BlockSpec((rows, _V), lambda i: (i, 0))],
        out_specs=pl.BlockSpec((1, 1), lambda i: (0, 0)),
        out_shape=jax.ShapeDtypeStruct((1, 1), jnp.float32),
    )(lg2)

    bik, t2k = pl.pallas_call(
        _prep_body,
        out_shape=[jax.ShapeDtypeStruct((_B, 208), jnp.int32),
                   jax.ShapeDtypeStruct((_B, 208), jnp.int32)],
    )(toks)
    bik3 = bik.reshape(_B, _NCH, 16)
    t2k3 = t2k.reshape(_B, _NCH, 16)

    tbl_bi = _sc_bi_scatter(bik3)
    tbl_tri, part_bi, keys3 = _sc_tri_scatter(bik3, t2k3, tbl_bi)
    tbl_pr, part_tri, keys4 = _sc_pr_scatter(keys3, tbl_tri)
    part_pr = _sc_pr_gather(keys4, tbl_pr)

    out = pl.pallas_call(
        _stats_body,
        out_shape=jax.ShapeDtypeStruct((1, 6), jnp.float32),
        scratch_shapes=[pltpu.VMEM((1, _V), jnp.float32),
                        pltpu.VMEM((_B, _V), jnp.float32)],
    )(toks, part_bi, part_tri, part_pr, conf)
    return out.reshape(6)


def kernel(generated_tokens, generated_logits, vocab_size):
    return _run(generated_tokens, generated_logits)
